# Initial kernel scaffold; baseline (speedup 1.0000x reference)
import jax
import jax.numpy as jnp
from jax.experimental import pallas as pl


def kernel(x, edge_index, edge_type, W_rel1, W_root1, b1, W_rel2, W_root2, b2):
    raise NotImplementedError("write your pallas kernel here")


# speedup vs baseline: 14.6324x; 14.6324x over previous
import functools

import jax
import jax.numpy as jnp
from jax import lax
from jax.experimental import pallas as pl
from jax.experimental.pallas import tpu as pltpu
from jax.experimental.pallas import tpu_sc as plsc

N = 10000
E = 320000
D = 128
R = 8
RN = R * N
YR = (R + 1) * N

NS = 16
L = 16
CHUNK = 80
ROUND = 50
NROUND = 5

NPT = 632
NPT_FULL = NPT // CHUNK
NPT_REM = NPT - NPT_FULL * CHUNK

_mesh1 = plsc.VectorSubcoreMesh(core_axis_name="c", subcore_axis_name="s",
                                num_cores=1)
_sc_params = pltpu.CompilerParams(needs_layout_passes=False)

_CROWS = RN // D
_ZR = 40


def _count_body(ti_hbm, di_hbm, cnt_out, cnt_tab, zbuf, tbs, dbs, kbs, colb,
                pay0, pay1, sem0, sem1):
    sid = lax.axis_index("s")
    z16 = jnp.zeros((L,), jnp.float32)
    i16 = lax.iota(jnp.int32, L)
    pays = (pay0, pay1)
    sems = (sem0, sem1)

    def zrow(i, _):
        for jx in range(D // L):
            zbuf[i, pl.ds(jx * L, L)] = z16
        return 0
    lax.fori_loop(0, _ZR, zrow, 0)
    zbase = jnp.minimum(sid * _ZR, _CROWS - _ZR)
    pltpu.sync_copy(zbuf, cnt_tab.at[pl.ds(zbase, _ZR)])

    plsc.subcore_barrier()

    def build(j, pay):
        for v in range(CHUNK // L):
            t = tbs[j, pl.ds(v * L, L)]
            d = dbs[j, pl.ds(v * L, L)]
            k = t * N + d
            kbs[j, pl.ds(v * L, L)] = lax.shift_right_logical(k, 7)
            colb[pl.ds(v * L, L)] = lax.bitwise_and(k, 127)

        def prow(e, _):
            c = plsc.load_gather(colb, [jnp.full((L,), e, jnp.int32)])
            for jx in range(D // L):
                pay[e, pl.ds(jx * L, L)] = jnp.where(
                    i16 + (jx * L) == c, jnp.float32(1.0), jnp.float32(0.0))
            return 0
        lax.fori_loop(0, CHUNK, prow, 0)

    def start_s(j, b):
        pltpu.async_copy(pays[b], cnt_tab.at[kbs.at[j]], sems[b], add=True)

    def wait_s(j, b):
        pltpu.make_async_copy(pays[b], cnt_tab.at[kbs.at[j]], sems[b]).wait()

    def round_body(r, _):
        pltpu.sync_copy(ti_hbm.at[sid, r], tbs)
        pltpu.sync_copy(di_hbm.at[sid, r], dbs)
        for b in range(2):
            build(b, pays[b])
            start_s(b, b)

        def pair(jj, _):
            for b in range(2):
                j = 2 * jj + 2 + b
                wait_s(j - 2, b)
                build(j, pays[b])
                start_s(j, b)
            return 0
        lax.fori_loop(0, ROUND // 2 - 1, pair, 0)
        wait_s(ROUND - 2, 0)
        wait_s(ROUND - 1, 1)
        return 0
    lax.fori_loop(0, NROUND, round_body, 0)

    plsc.subcore_barrier()

    @pl.when(sid == 0)
    def _():
        pltpu.sync_copy(cnt_tab, cnt_out)


_count_call = pl.kernel(
    _count_body,
    out_type=(),
    mesh=_mesh1,
    compiler_params=_sc_params,
    scratch_types=[
        pltpu.VMEM_SHARED((_CROWS, D), jnp.float32),
        pltpu.VMEM((_ZR, D), jnp.float32),
        pltpu.VMEM((ROUND, CHUNK), jnp.int32),
        pltpu.VMEM((ROUND, CHUNK), jnp.int32),
        pltpu.VMEM((ROUND, CHUNK), jnp.int32),
        pltpu.VMEM((CHUNK,), jnp.int32),
        pltpu.VMEM((CHUNK, D), jnp.float32),
        pltpu.VMEM((CHUNK, D), jnp.float32),
        pltpu.SemaphoreType.DMA,
        pltpu.SemaphoreType.DMA,
    ],
)


@jax.jit
def _count(ti4, di4):
    c = jax.new_ref(jnp.zeros((_CROWS, D), jnp.float32))
    _count_call(ti4, di4, c)
    return c[...]


def _scale_body(cnt_hbm, ti_hbm, si_hbm, di_hbm, g_out, s_out,
                cntt, tbs, sbs, dbs, gbuf, sbuf):
    sid = lax.axis_index("s")
    one = jnp.full((L,), 1.0, jnp.float32)

    pltpu.sync_copy(cnt_hbm, cntt)

    def round_body(r, _):
        pltpu.sync_copy(ti_hbm.at[sid, r], tbs)
        pltpu.sync_copy(si_hbm.at[sid, r], sbs)
        pltpu.sync_copy(di_hbm.at[sid, r], dbs)

        def chunk_body(j, _):
            for v in range(CHUNK // L):
                t = tbs[j, pl.ds(v * L, L)]
                s = sbs[j, pl.ds(v * L, L)]
                d = dbs[j, pl.ds(v * L, L)]
                gbuf[j, pl.ds(v * L, L)] = (t + 1) * N + s
                c = plsc.load_gather(cntt, [t * N + d])
                sbuf[j, pl.ds(v * L, L)] = one / jnp.maximum(c, one)
            return 0
        lax.fori_loop(0, ROUND, chunk_body, 0)
        pltpu.sync_copy(gbuf, g_out.at[sid, r])
        pltpu.sync_copy(sbuf, s_out.at[sid, r])
        return 0
    lax.fori_loop(0, NROUND, round_body, 0)


_scale_call = pl.kernel(
    _scale_body,
    out_type=(),
    mesh=_mesh1,
    compiler_params=_sc_params,
    scratch_types=[
        pltpu.VMEM((RN,), jnp.float32),
        pltpu.VMEM((ROUND, CHUNK), jnp.int32),
        pltpu.VMEM((ROUND, CHUNK), jnp.int32),
        pltpu.VMEM((ROUND, CHUNK), jnp.int32),
        pltpu.VMEM((ROUND, CHUNK), jnp.int32),
        pltpu.VMEM((ROUND, CHUNK), jnp.float32),
    ],
)


@jax.jit
def _edge_scale(cntf, ti4, si4, di4):
    g = jax.new_ref(jnp.zeros((NS, NROUND, ROUND, CHUNK), jnp.int32))
    s = jax.new_ref(jnp.zeros((NS, NROUND, ROUND, CHUNK), jnp.float32))
    _scale_call(cntf, ti4, si4, di4, g, s)
    return g[...], s[...]


def _scatter_body(y_hbm, g_hbm, d_hbm, s_hbm, part_out,
                  acc, gbs, dbs, ssb, sv0, sv1, rows0, rows1,
                  sg0, sg1, ss0, ss1):
    sid = lax.axis_index("s")
    z16 = jnp.zeros((L,), jnp.float32)
    abase = jnp.minimum(sid * NPT, N - NPT)
    rows = (rows0, rows1)
    svs = (sv0, sv1)
    sgs = (sg0, sg1)
    sss = (ss0, ss1)

    def zrow(i, _):
        for j in range(D // L):
            rows0[i, pl.ds(j * L, L)] = z16
        return 0
    lax.fori_loop(0, CHUNK, zrow, 0)

    def zacc(j, _):
        pltpu.sync_copy(rows0, acc.at[pl.ds(abase + j * CHUNK, CHUNK)])
        return 0
    lax.fori_loop(0, NPT_FULL, zacc, 0)
    pltpu.sync_copy(rows0.at[pl.ds(0, NPT_REM)],
                    acc.at[pl.ds(abase + NPT_FULL * CHUNK, NPT_REM)])

    plsc.subcore_barrier()

    def prep(j, b):
        for v in range(CHUNK // L):
            svs[b][pl.ds(v * L, L)] = ssb[j, pl.ds(v * L, L)]

    def start_g(j, b):
        pltpu.async_copy(y_hbm.at[gbs.at[j]], rows[b], sgs[b])

    def wait_g(j, b):
        pltpu.make_async_copy(y_hbm.at[gbs.at[j]], rows[b], sgs[b]).wait()

    def start_s(j, b):
        pltpu.async_copy(rows[b], acc.at[dbs.at[j]], sss[b], add=True)

    def wait_s(j, b):
        pltpu.make_async_copy(rows[b], acc.at[dbs.at[j]], sss[b]).wait()

    def round_body(r, _):
        pltpu.sync_copy(g_hbm.at[sid, r], gbs)
        pltpu.sync_copy(d_hbm.at[sid, r], dbs)
        pltpu.sync_copy(s_hbm.at[sid, r], ssb)
        for b in range(2):
            prep(b, b)
            start_g(b, b)

        def pair(jj, _):
            for b in range(2):
                j = 2 * jj + b
                wait_g(j, b)

                def scale(e, _):
                    sc = plsc.load_gather(svs[b], [jnp.full((L,), e, jnp.int32)])
                    for jx in range(D // L):
                        rows[b][e, pl.ds(jx * L, L)] = (
                            rows[b][e, pl.ds(jx * L, L)] * sc)
                    return 0
                lax.fori_loop(0, CHUNK, scale, 0)

                start_s(j, b)
                wait_s(j, b)
                jn = jnp.minimum(j + 2, ROUND - 1)
                prep(jn, b)
                start_g(jn, b)
            return 0
        lax.fori_loop(0, ROUND // 2, pair, 0)
        wait_g(ROUND - 1, 0)
        wait_g(ROUND - 1, 1)
        return 0
    lax.fori_loop(0, NROUND, round_body, 0)

    plsc.subcore_barrier()

    pltpu.sync_copy(acc.at[pl.ds(abase, NPT)], part_out.at[pl.ds(abase, NPT)])


_scatter_call = pl.kernel(
    _scatter_body,
    out_type=(),
    mesh=_mesh1,
    compiler_params=_sc_params,
    scratch_types=[
        pltpu.VMEM_SHARED((N, D), jnp.float32),
        pltpu.VMEM((ROUND, CHUNK), jnp.int32),
        pltpu.VMEM((ROUND, CHUNK), jnp.int32),
        pltpu.VMEM((ROUND, CHUNK), jnp.float32),
        pltpu.VMEM((CHUNK,), jnp.float32),
        pltpu.VMEM((CHUNK,), jnp.float32),
        pltpu.VMEM((CHUNK, D), jnp.float32),
        pltpu.VMEM((CHUNK, D), jnp.float32),
        pltpu.SemaphoreType.DMA,
        pltpu.SemaphoreType.DMA,
        pltpu.SemaphoreType.DMA,
        pltpu.SemaphoreType.DMA,
    ],
)


@jax.jit
def _scatter(ycat, g4, d4, s4):
    o = jax.new_ref(jnp.zeros((N, D), jnp.float32))
    _scatter_call(ycat, g4, d4, s4, o)
    return o[...]


_BT = 2000


def _transform_kernel(x_ref, w_ref, b_ref, y_ref):
    i = pl.program_id(1)
    y = jnp.dot(x_ref[...], w_ref[0],
                preferred_element_type=jnp.float32,
                precision=lax.Precision.HIGHEST)
    scale = jnp.where(i == 0, jnp.float32(1.0), jnp.float32(0.0))
    y_ref[0] = y + b_ref[...] * scale


@jax.jit
def _transform(x, wcat, b2d):
    return pl.pallas_call(
        _transform_kernel,
        grid=(N // _BT, R + 1),
        in_specs=[
            pl.BlockSpec((_BT, D), lambda nb, i: (nb, 0)),
            pl.BlockSpec((1, D, D), lambda nb, i: (i, 0, 0)),
            pl.BlockSpec((1, D), lambda nb, i: (0, 0)),
        ],
        out_specs=pl.BlockSpec((1, _BT, D), lambda nb, i: (i, nb, 0)),
        out_shape=jax.ShapeDtypeStruct((R + 1, N, D), jnp.float32),
    )(x, wcat, b2d)


def _combine_kernel(y_ref, p_ref, o_ref, *, relu):
    o = y_ref[0] + p_ref[...]
    if relu:
        o = jnp.maximum(o, 0.0)
    o_ref[...] = o


def _make_combine(relu):
    f = pl.pallas_call(
        functools.partial(_combine_kernel, relu=relu),
        grid=(N // _BT,),
        in_specs=[
            pl.BlockSpec((1, _BT, D), lambda nb: (0, nb, 0)),
            pl.BlockSpec((_BT, D), lambda nb: (nb, 0)),
        ],
        out_specs=pl.BlockSpec((_BT, D), lambda nb: (nb, 0)),
        out_shape=jax.ShapeDtypeStruct((N, D), jnp.float32),
    )
    return jax.jit(f)


_combine_relu = _make_combine(True)
_combine_plain = _make_combine(False)


def kernel(x, edge_index, edge_type, W_rel1, W_root1, b1, W_rel2, W_root2, b2):
    si4 = edge_index[0].reshape(NS, NROUND, ROUND, CHUNK)
    di4 = edge_index[1].reshape(NS, NROUND, ROUND, CHUNK)
    ti4 = edge_type.reshape(NS, NROUND, ROUND, CHUNK)

    cntf = _count(ti4, di4).reshape(RN)
    g4, s4 = _edge_scale(cntf, ti4, si4, di4)

    wcat1 = jnp.concatenate([W_root1[None], W_rel1], axis=0)
    y1 = _transform(x, wcat1, b1.reshape(1, D))
    p1 = _scatter(y1.reshape(YR, D), g4, di4, s4)
    h = _combine_relu(y1, p1)

    wcat2 = jnp.concatenate([W_root2[None], W_rel2], axis=0)
    y2 = _transform(h, wcat2, b2.reshape(1, D))
    p2 = _scatter(y2.reshape(YR, D), g4, di4, s4)
    return _combine_plain(y2, p2)


# speedup vs baseline: 16.2381x; 1.1097x over previous
import functools

import jax
import jax.numpy as jnp
from jax import lax
from jax.experimental import pallas as pl
from jax.experimental.pallas import tpu as pltpu
from jax.experimental.pallas import tpu_sc as plsc

N = 10000
E = 320000
D = 128
R = 8
RN = R * N
YR = (R + 1) * N

NS = 16
L = 16
CHUNK = 80
ROUND = 25
NROUND = 10

NPT = 632
NPT_FULL = NPT // CHUNK
NPT_REM = NPT - NPT_FULL * CHUNK

_mesh1 = plsc.VectorSubcoreMesh(core_axis_name="c", subcore_axis_name="s",
                                num_cores=1)
_sc_params = pltpu.CompilerParams(needs_layout_passes=False)

_CROWS = RN // D
_ZR = 40


def _count_body(ti_hbm, di_hbm, cnt_out, cnt_tab, zbuf, tbs, dbs, kbs, colb,
                pay0, pay1, sem0, sem1):
    sid = lax.axis_index("s")
    z16 = jnp.zeros((L,), jnp.float32)
    i16 = lax.iota(jnp.int32, L)
    pays = (pay0, pay1)
    sems = (sem0, sem1)

    def zrow(i, _):
        for jx in range(D // L):
            zbuf[i, pl.ds(jx * L, L)] = z16
        return 0
    lax.fori_loop(0, _ZR, zrow, 0)
    zbase = jnp.minimum(sid * _ZR, _CROWS - _ZR)
    pltpu.sync_copy(zbuf, cnt_tab.at[pl.ds(zbase, _ZR)])

    plsc.subcore_barrier()

    def build(j, pay):
        for v in range(CHUNK // L):
            t = tbs[j, pl.ds(v * L, L)]
            d = dbs[j, pl.ds(v * L, L)]
            k = t * N + d
            kbs[j, pl.ds(v * L, L)] = lax.shift_right_logical(k, 7)
            colb[pl.ds(v * L, L)] = lax.bitwise_and(k, 127)

        def prow(e, _):
            c = plsc.load_gather(colb, [jnp.full((L,), e, jnp.int32)])
            for jx in range(D // L):
                pay[e, pl.ds(jx * L, L)] = jnp.where(
                    i16 + (jx * L) == c, jnp.float32(1.0), jnp.float32(0.0))
            return 0
        lax.fori_loop(0, CHUNK, prow, 0)

    def start_s(j, b):
        pltpu.async_copy(pays[b], cnt_tab.at[kbs.at[j]], sems[b], add=True)

    def wait_s(j, b):
        pltpu.make_async_copy(pays[b], cnt_tab.at[kbs.at[j]], sems[b]).wait()

    def round_body(r, _):
        pltpu.sync_copy(ti_hbm.at[sid, r], tbs)
        pltpu.sync_copy(di_hbm.at[sid, r], dbs)
        for b in range(2):
            build(b, pays[b])
            start_s(b, b)

        def pair(jj, _):
            for b in range(2):
                j = 2 * jj + 2 + b
                wait_s(j - 2, b)
                build(j, pays[b])
                start_s(j, b)
            return 0
        lax.fori_loop(0, (ROUND - 3) // 2, pair, 0)
        wait_s(ROUND - 3, 0)
        build(ROUND - 1, pays[0])
        start_s(ROUND - 1, 0)
        wait_s(ROUND - 2, 1)
        wait_s(ROUND - 1, 0)
        return 0
    lax.fori_loop(0, NROUND, round_body, 0)

    plsc.subcore_barrier()

    @pl.when(sid == 0)
    def _():
        pltpu.sync_copy(cnt_tab, cnt_out)


_count_call = pl.kernel(
    _count_body,
    out_type=(),
    mesh=_mesh1,
    compiler_params=_sc_params,
    scratch_types=[
        pltpu.VMEM_SHARED((_CROWS, D), jnp.float32),
        pltpu.VMEM((_ZR, D), jnp.float32),
        pltpu.VMEM((ROUND, CHUNK), jnp.int32),
        pltpu.VMEM((ROUND, CHUNK), jnp.int32),
        pltpu.VMEM((ROUND, CHUNK), jnp.int32),
        pltpu.VMEM((CHUNK,), jnp.int32),
        pltpu.VMEM((CHUNK, D), jnp.float32),
        pltpu.VMEM((CHUNK, D), jnp.float32),
        pltpu.SemaphoreType.DMA,
        pltpu.SemaphoreType.DMA,
    ],
)


@jax.jit
def _count(ti4, di4):
    c = jax.new_ref(jnp.zeros((_CROWS, D), jnp.float32))
    _count_call(ti4, di4, c)
    return c[...]


def _scale_body(cnt_hbm, ti_hbm, si_hbm, di_hbm, g_out, s_out,
                cntt, tbs, sbs, dbs, gbuf, sbuf):
    sid = lax.axis_index("s")
    one = jnp.full((L,), 1.0, jnp.float32)

    pltpu.sync_copy(cnt_hbm, cntt)

    def round_body(r, _):
        pltpu.sync_copy(ti_hbm.at[sid, r], tbs)
        pltpu.sync_copy(si_hbm.at[sid, r], sbs)
        pltpu.sync_copy(di_hbm.at[sid, r], dbs)

        def chunk_body(j, _):
            for v in range(CHUNK // L):
                t = tbs[j, pl.ds(v * L, L)]
                s = sbs[j, pl.ds(v * L, L)]
                d = dbs[j, pl.ds(v * L, L)]
                gbuf[j, pl.ds(v * L, L)] = (t + 1) * N + s
                c = plsc.load_gather(cntt, [t * N + d])
                sbuf[j, pl.ds(v * L, L)] = one / jnp.maximum(c, one)
            return 0
        lax.fori_loop(0, ROUND, chunk_body, 0)
        pltpu.sync_copy(gbuf, g_out.at[sid, r])
        pltpu.sync_copy(sbuf, s_out.at[sid, r])
        return 0
    lax.fori_loop(0, NROUND, round_body, 0)


_scale_call = pl.kernel(
    _scale_body,
    out_type=(),
    mesh=_mesh1,
    compiler_params=_sc_params,
    scratch_types=[
        pltpu.VMEM((RN,), jnp.float32),
        pltpu.VMEM((ROUND, CHUNK), jnp.int32),
        pltpu.VMEM((ROUND, CHUNK), jnp.int32),
        pltpu.VMEM((ROUND, CHUNK), jnp.int32),
        pltpu.VMEM((ROUND, CHUNK), jnp.int32),
        pltpu.VMEM((ROUND, CHUNK), jnp.float32),
    ],
)


@jax.jit
def _edge_scale(cntf, ti4, si4, di4):
    g = jax.new_ref(jnp.zeros((NS, NROUND, ROUND, CHUNK), jnp.int32))
    s = jax.new_ref(jnp.zeros((NS, NROUND, ROUND, CHUNK), jnp.float32))
    _scale_call(cntf, ti4, si4, di4, g, s)
    return g[...], s[...]


def _scatter_body(y_hbm, g_hbm, d_hbm, s_hbm, part_out,
                  acc, gbs, dbs, ssb, sv0, sv1, sv2, rows0, rows1, rows2,
                  sg0, sg1, sg2, ss0, ss1, ss2):
    sid = lax.axis_index("s")
    z16 = jnp.zeros((L,), jnp.float32)
    abase = jnp.minimum(sid * NPT, N - NPT)
    rows = (rows0, rows1, rows2)
    svs = (sv0, sv1, sv2)
    sgs = (sg0, sg1, sg2)
    sss = (ss0, ss1, ss2)

    def zrow(i, _):
        for j in range(D // L):
            rows0[i, pl.ds(j * L, L)] = z16
        return 0
    lax.fori_loop(0, CHUNK, zrow, 0)

    def zacc(j, _):
        pltpu.sync_copy(rows0, acc.at[pl.ds(abase + j * CHUNK, CHUNK)])
        return 0
    lax.fori_loop(0, NPT_FULL, zacc, 0)
    pltpu.sync_copy(rows0.at[pl.ds(0, NPT_REM)],
                    acc.at[pl.ds(abase + NPT_FULL * CHUNK, NPT_REM)])

    plsc.subcore_barrier()

    def prep(j, b):
        for v in range(CHUNK // L):
            svs[b][pl.ds(v * L, L)] = ssb[j, pl.ds(v * L, L)]

    def start_g(j, b):
        pltpu.async_copy(y_hbm.at[gbs.at[j]], rows[b], sgs[b])

    def wait_g(j, b):
        pltpu.make_async_copy(y_hbm.at[gbs.at[j]], rows[b], sgs[b]).wait()

    def start_s(j, b):
        pltpu.async_copy(rows[b], acc.at[dbs.at[j]], sss[b], add=True)

    def wait_s(j, b):
        pltpu.make_async_copy(rows[b], acc.at[dbs.at[j]], sss[b]).wait()

    def scale(b):
        def body(e, _):
            sc = plsc.load_gather(svs[b], [jnp.full((L,), e, jnp.int32)])
            for jx in range(D // L):
                rows[b][e, pl.ds(jx * L, L)] = rows[b][e, pl.ds(jx * L, L)] * sc
            return 0
        lax.fori_loop(0, CHUNK, body, 0)

    def step(j, with_wait, with_prefetch):
        b = j % 3 if isinstance(j, int) else None
        wait_g(j, b)
        scale(b)
        start_s(j, b)
        if with_prefetch:
            b2 = (j + 2) % 3
            if with_wait:
                wait_s(j - 1, b2)
            prep(j + 2, b2)
            start_g(j + 2, b2)

    def round_body(r, _):
        pltpu.sync_copy(g_hbm.at[sid, r], gbs)
        pltpu.sync_copy(d_hbm.at[sid, r], dbs)
        pltpu.sync_copy(s_hbm.at[sid, r], ssb)
        prep(0, 0)
        start_g(0, 0)
        prep(1, 1)
        start_g(1, 1)
        step(0, False, True)
        for jp in range(1, 5):
            step(jp, True, True)

        def triple(g, _):
            for idx in range(3):
                j = 3 * g + 5 + idx
                b = (5 + idx) % 3
                wait_g(j, b)
                scale(b)
                start_s(j, b)
                b2 = (j0b := (5 + idx + 2) % 3)
                wait_s(j - 1, b2)
                prep(j + 2, b2)
                start_g(j + 2, b2)
            return 0
        lax.fori_loop(0, (ROUND - 7) // 3, triple, 0)
        step(ROUND - 2, False, False)
        step(ROUND - 1, False, False)
        wait_s(ROUND - 3, (ROUND - 3) % 3)
        wait_s(ROUND - 2, (ROUND - 2) % 3)
        wait_s(ROUND - 1, (ROUND - 1) % 3)
        return 0
    lax.fori_loop(0, NROUND, round_body, 0)

    plsc.subcore_barrier()

    pltpu.sync_copy(acc.at[pl.ds(abase, NPT)], part_out.at[pl.ds(abase, NPT)])


_scatter_call = pl.kernel(
    _scatter_body,
    out_type=(),
    mesh=_mesh1,
    compiler_params=_sc_params,
    scratch_types=[
        pltpu.VMEM_SHARED((N, D), jnp.float32),
        pltpu.VMEM((ROUND, CHUNK), jnp.int32),
        pltpu.VMEM((ROUND, CHUNK), jnp.int32),
        pltpu.VMEM((ROUND, CHUNK), jnp.float32),
        pltpu.VMEM((CHUNK,), jnp.float32),
        pltpu.VMEM((CHUNK,), jnp.float32),
        pltpu.VMEM((CHUNK,), jnp.float32),
        pltpu.VMEM((CHUNK, D), jnp.float32),
        pltpu.VMEM((CHUNK, D), jnp.float32),
        pltpu.VMEM((CHUNK, D), jnp.float32),
        pltpu.SemaphoreType.DMA,
        pltpu.SemaphoreType.DMA,
        pltpu.SemaphoreType.DMA,
        pltpu.SemaphoreType.DMA,
        pltpu.SemaphoreType.DMA,
        pltpu.SemaphoreType.DMA,
    ],
)


@jax.jit
def _scatter(ycat, g4, d4, s4):
    o = jax.new_ref(jnp.zeros((N, D), jnp.float32))
    _scatter_call(ycat, g4, d4, s4, o)
    return o[...]


_BT = 2000


def _transform_kernel(x_ref, w_ref, b_ref, y_ref):
    i = pl.program_id(1)
    y = jnp.dot(x_ref[...], w_ref[0],
                preferred_element_type=jnp.float32,
                precision=lax.Precision.HIGHEST)
    scale = jnp.where(i == 0, jnp.float32(1.0), jnp.float32(0.0))
    y_ref[0] = y + b_ref[...] * scale


@jax.jit
def _transform(x, wcat, b2d):
    return pl.pallas_call(
        _transform_kernel,
        grid=(N // _BT, R + 1),
        in_specs=[
            pl.BlockSpec((_BT, D), lambda nb, i: (nb, 0)),
            pl.BlockSpec((1, D, D), lambda nb, i: (i, 0, 0)),
            pl.BlockSpec((1, D), lambda nb, i: (0, 0)),
        ],
        out_specs=pl.BlockSpec((1, _BT, D), lambda nb, i: (i, nb, 0)),
        out_shape=jax.ShapeDtypeStruct((R + 1, N, D), jnp.float32),
    )(x, wcat, b2d)


def _combine_kernel(y_ref, p_ref, o_ref, *, relu):
    o = y_ref[0] + p_ref[...]
    if relu:
        o = jnp.maximum(o, 0.0)
    o_ref[...] = o


def _make_combine(relu):
    f = pl.pallas_call(
        functools.partial(_combine_kernel, relu=relu),
        grid=(N // _BT,),
        in_specs=[
            pl.BlockSpec((1, _BT, D), lambda nb: (0, nb, 0)),
            pl.BlockSpec((_BT, D), lambda nb: (nb, 0)),
        ],
        out_specs=pl.BlockSpec((_BT, D), lambda nb: (nb, 0)),
        out_shape=jax.ShapeDtypeStruct((N, D), jnp.float32),
    )
    return jax.jit(f)


_combine_relu = _make_combine(True)
_combine_plain = _make_combine(False)


def kernel(x, edge_index, edge_type, W_rel1, W_root1, b1, W_rel2, W_root2, b2):
    si4 = edge_index[0].reshape(NS, NROUND, ROUND, CHUNK)
    di4 = edge_index[1].reshape(NS, NROUND, ROUND, CHUNK)
    ti4 = edge_type.reshape(NS, NROUND, ROUND, CHUNK)

    cntf = _count(ti4, di4).reshape(RN)
    g4, s4 = _edge_scale(cntf, ti4, si4, di4)

    wcat1 = jnp.concatenate([W_root1[None], W_rel1], axis=0)
    y1 = _transform(x, wcat1, b1.reshape(1, D))
    p1 = _scatter(y1.reshape(YR, D), g4, di4, s4)
    h = _combine_relu(y1, p1)

    wcat2 = jnp.concatenate([W_root2[None], W_rel2], axis=0)
    y2 = _transform(h, wcat2, b2.reshape(1, D))
    p2 = _scatter(y2.reshape(YR, D), g4, di4, s4)
    return _combine_plain(y2, p2)


# speedup vs baseline: 18.0906x; 1.1141x over previous
import functools

import jax
import jax.numpy as jnp
from jax import lax
from jax.experimental import pallas as pl
from jax.experimental.pallas import tpu as pltpu
from jax.experimental.pallas import tpu_sc as plsc

N = 10000
E = 320000
D = 128
R = 8
RN = R * N
YR = (R + 1) * N

NC = 2
NS = 16
L = 16
CHUNK = 80
ROUND = 25
NROUND = 10

NPT = 632
NPT_FULL = NPT // CHUNK
NPT_REM = NPT - NPT_FULL * CHUNK

_mesh1 = plsc.VectorSubcoreMesh(core_axis_name="c", subcore_axis_name="s",
                                num_cores=1)
_mesh2 = plsc.VectorSubcoreMesh(core_axis_name="c", subcore_axis_name="s")
_sc_params = pltpu.CompilerParams(needs_layout_passes=False)

_CROWS = RN // D
_ZR = 40


def _count_body(ti_hbm, di_hbm, cnt_out, cnt_tab, zbuf, tbs, dbs, kbs, colb,
                pay0, pay1, sem0, sem1):
    core = lax.axis_index("c")
    sid = lax.axis_index("s")
    z16 = jnp.zeros((L,), jnp.float32)
    i16 = lax.iota(jnp.int32, L)
    pays = (pay0, pay1)
    sems = (sem0, sem1)

    def zrow(i, _):
        for jx in range(D // L):
            zbuf[i, pl.ds(jx * L, L)] = z16
        return 0
    lax.fori_loop(0, _ZR, zrow, 0)
    zbase = jnp.minimum(sid * _ZR, _CROWS - _ZR)
    pltpu.sync_copy(zbuf, cnt_tab.at[pl.ds(zbase, _ZR)])

    plsc.subcore_barrier()

    def build(j, pay):
        for v in range(CHUNK // L):
            t = tbs[j, pl.ds(v * L, L)]
            d = dbs[j, pl.ds(v * L, L)]
            k = t * N + d
            kbs[j, pl.ds(v * L, L)] = lax.shift_right_logical(k, 7)
            colb[pl.ds(v * L, L)] = lax.bitwise_and(k, 127)

        def prow(e, _):
            c = plsc.load_gather(colb, [jnp.full((L,), e, jnp.int32)])
            for jx in range(D // L):
                pay[e, pl.ds(jx * L, L)] = jnp.where(
                    i16 + (jx * L) == c, jnp.float32(1.0), jnp.float32(0.0))
            return 0
        lax.fori_loop(0, CHUNK, prow, 0)

    def start_s(j, b):
        pltpu.async_copy(pays[b], cnt_tab.at[kbs.at[j]], sems[b], add=True)

    def wait_s(j, b):
        pltpu.make_async_copy(pays[b], cnt_tab.at[kbs.at[j]], sems[b]).wait()

    def round_body(r, _):
        rr = core * (NROUND // 2) + r
        pltpu.sync_copy(ti_hbm.at[sid, rr], tbs)
        pltpu.sync_copy(di_hbm.at[sid, rr], dbs)
        for b in range(2):
            build(b, pays[b])
            start_s(b, b)

        def pair(jj, _):
            for b in range(2):
                j = 2 * jj + 2 + b
                wait_s(j - 2, b)
                build(j, pays[b])
                start_s(j, b)
            return 0
        lax.fori_loop(0, (ROUND - 3) // 2, pair, 0)
        wait_s(ROUND - 3, 0)
        build(ROUND - 1, pays[0])
        start_s(ROUND - 1, 0)
        wait_s(ROUND - 2, 1)
        wait_s(ROUND - 1, 0)
        return 0
    lax.fori_loop(0, NROUND // 2, round_body, 0)

    plsc.subcore_barrier()

    @pl.when(sid == 0)
    def _():
        pltpu.sync_copy(cnt_tab, cnt_out.at[core])


_count_call = pl.kernel(
    _count_body,
    out_type=(),
    mesh=_mesh2,
    compiler_params=_sc_params,
    scratch_types=[
        pltpu.VMEM_SHARED((_CROWS, D), jnp.float32),
        pltpu.VMEM((_ZR, D), jnp.float32),
        pltpu.VMEM((ROUND, CHUNK), jnp.int32),
        pltpu.VMEM((ROUND, CHUNK), jnp.int32),
        pltpu.VMEM((ROUND, CHUNK), jnp.int32),
        pltpu.VMEM((CHUNK,), jnp.int32),
        pltpu.VMEM((CHUNK, D), jnp.float32),
        pltpu.VMEM((CHUNK, D), jnp.float32),
        pltpu.SemaphoreType.DMA,
        pltpu.SemaphoreType.DMA,
    ],
)


@jax.jit
def _count(ti4, di4):
    c = jax.new_ref(jnp.zeros((NC, _CROWS, D), jnp.float32))
    _count_call(ti4, di4, c)
    return c[...]


def _scale_body(cnt_hbm, ti_hbm, si_hbm, di_hbm, g_out, s_out,
                cntt, tbs, sbs, dbs, gbuf, sbuf):
    core = lax.axis_index("c")
    sid = lax.axis_index("s")
    one = jnp.full((L,), 1.0, jnp.float32)

    pltpu.sync_copy(cnt_hbm, cntt)

    def round_body(r, _):
        rr = core * (NROUND // 2) + r
        pltpu.sync_copy(ti_hbm.at[sid, rr], tbs)
        pltpu.sync_copy(si_hbm.at[sid, rr], sbs)
        pltpu.sync_copy(di_hbm.at[sid, rr], dbs)

        def chunk_body(j, _):
            for v in range(CHUNK // L):
                t = tbs[j, pl.ds(v * L, L)]
                s = sbs[j, pl.ds(v * L, L)]
                d = dbs[j, pl.ds(v * L, L)]
                gbuf[j, pl.ds(v * L, L)] = (t + 1) * N + s
                c = plsc.load_gather(cntt, [t * N + d])
                sbuf[j, pl.ds(v * L, L)] = one / jnp.maximum(c, one)
            return 0
        lax.fori_loop(0, ROUND, chunk_body, 0)
        pltpu.sync_copy(gbuf, g_out.at[sid, rr])
        pltpu.sync_copy(sbuf, s_out.at[sid, rr])
        return 0
    lax.fori_loop(0, NROUND // 2, round_body, 0)


_scale_call = pl.kernel(
    _scale_body,
    out_type=(),
    mesh=_mesh2,
    compiler_params=_sc_params,
    scratch_types=[
        pltpu.VMEM((RN,), jnp.float32),
        pltpu.VMEM((ROUND, CHUNK), jnp.int32),
        pltpu.VMEM((ROUND, CHUNK), jnp.int32),
        pltpu.VMEM((ROUND, CHUNK), jnp.int32),
        pltpu.VMEM((ROUND, CHUNK), jnp.int32),
        pltpu.VMEM((ROUND, CHUNK), jnp.float32),
    ],
)


@jax.jit
def _edge_scale(cntf, ti4, si4, di4):
    g = jax.new_ref(jnp.zeros((NS, NROUND, ROUND, CHUNK), jnp.int32))
    s = jax.new_ref(jnp.zeros((NS, NROUND, ROUND, CHUNK), jnp.float32))
    _scale_call(cntf, ti4, si4, di4, g, s)
    return g[...], s[...]


def _scatter_body(y_hbm, g_hbm, d_hbm, s_hbm, part_out,
                  acc, gbs, dbs, ssb, sv0, sv1, sv2, rows0, rows1, rows2,
                  sg0, sg1, sg2, ss0, ss1, ss2):
    sid = lax.axis_index("s")
    z16 = jnp.zeros((L,), jnp.float32)
    abase = jnp.minimum(sid * NPT, N - NPT)
    rows = (rows0, rows1, rows2)
    svs = (sv0, sv1, sv2)
    sgs = (sg0, sg1, sg2)
    sss = (ss0, ss1, ss2)

    def zrow(i, _):
        for j in range(D // L):
            rows0[i, pl.ds(j * L, L)] = z16
        return 0
    lax.fori_loop(0, CHUNK, zrow, 0)

    def zacc(j, _):
        pltpu.sync_copy(rows0, acc.at[pl.ds(abase + j * CHUNK, CHUNK)])
        return 0
    lax.fori_loop(0, NPT_FULL, zacc, 0)
    pltpu.sync_copy(rows0.at[pl.ds(0, NPT_REM)],
                    acc.at[pl.ds(abase + NPT_FULL * CHUNK, NPT_REM)])

    plsc.subcore_barrier()

    def prep(j, b):
        for v in range(CHUNK // L):
            svs[b][pl.ds(v * L, L)] = ssb[j, pl.ds(v * L, L)]

    def start_g(j, b):
        pltpu.async_copy(y_hbm.at[gbs.at[j]], rows[b], sgs[b])

    def wait_g(j, b):
        pltpu.make_async_copy(y_hbm.at[gbs.at[j]], rows[b], sgs[b]).wait()

    def start_s(j, b):
        pltpu.async_copy(rows[b], acc.at[dbs.at[j]], sss[b], add=True)

    def wait_s(j, b):
        pltpu.make_async_copy(rows[b], acc.at[dbs.at[j]], sss[b]).wait()

    def scale(b):
        def body(e, _):
            sc = plsc.load_gather(svs[b], [jnp.full((L,), e, jnp.int32)])
            for jx in range(D // L):
                rows[b][e, pl.ds(jx * L, L)] = rows[b][e, pl.ds(jx * L, L)] * sc
            return 0
        lax.fori_loop(0, CHUNK, body, 0)

    def step(j, with_wait, with_prefetch):
        b = j % 3 if isinstance(j, int) else None
        wait_g(j, b)
        scale(b)
        start_s(j, b)
        if with_prefetch:
            b2 = (j + 2) % 3
            if with_wait:
                wait_s(j - 1, b2)
            prep(j + 2, b2)
            start_g(j + 2, b2)

    def round_body(r, _):
        pltpu.sync_copy(g_hbm.at[sid, r], gbs)
        pltpu.sync_copy(d_hbm.at[sid, r], dbs)
        pltpu.sync_copy(s_hbm.at[sid, r], ssb)
        prep(0, 0)
        start_g(0, 0)
        prep(1, 1)
        start_g(1, 1)
        step(0, False, True)
        for jp in range(1, 5):
            step(jp, True, True)

        def triple(g, _):
            for idx in range(3):
                j = 3 * g + 5 + idx
                b = (5 + idx) % 3
                wait_g(j, b)
                scale(b)
                start_s(j, b)
                b2 = (j0b := (5 + idx + 2) % 3)
                wait_s(j - 1, b2)
                prep(j + 2, b2)
                start_g(j + 2, b2)
            return 0
        lax.fori_loop(0, (ROUND - 7) // 3, triple, 0)
        step(ROUND - 2, False, False)
        step(ROUND - 1, False, False)
        wait_s(ROUND - 3, (ROUND - 3) % 3)
        wait_s(ROUND - 2, (ROUND - 2) % 3)
        wait_s(ROUND - 1, (ROUND - 1) % 3)
        return 0
    lax.fori_loop(0, NROUND, round_body, 0)

    plsc.subcore_barrier()

    pltpu.sync_copy(acc.at[pl.ds(abase, NPT)], part_out.at[pl.ds(abase, NPT)])


_scatter_call = pl.kernel(
    _scatter_body,
    out_type=(),
    mesh=_mesh1,
    compiler_params=_sc_params,
    scratch_types=[
        pltpu.VMEM_SHARED((N, D), jnp.float32),
        pltpu.VMEM((ROUND, CHUNK), jnp.int32),
        pltpu.VMEM((ROUND, CHUNK), jnp.int32),
        pltpu.VMEM((ROUND, CHUNK), jnp.float32),
        pltpu.VMEM((CHUNK,), jnp.float32),
        pltpu.VMEM((CHUNK,), jnp.float32),
        pltpu.VMEM((CHUNK,), jnp.float32),
        pltpu.VMEM((CHUNK, D), jnp.float32),
        pltpu.VMEM((CHUNK, D), jnp.float32),
        pltpu.VMEM((CHUNK, D), jnp.float32),
        pltpu.SemaphoreType.DMA,
        pltpu.SemaphoreType.DMA,
        pltpu.SemaphoreType.DMA,
        pltpu.SemaphoreType.DMA,
        pltpu.SemaphoreType.DMA,
        pltpu.SemaphoreType.DMA,
    ],
)


@jax.jit
def _scatter(ycat, g4, d4, s4):
    o = jax.new_ref(jnp.zeros((N, D), jnp.float32))
    _scatter_call(ycat, g4, d4, s4, o)
    return o[...]


def _sum_kernel(c_ref, o_ref):
    o_ref[...] = c_ref[0] + c_ref[1]


_sum_counts = jax.jit(pl.pallas_call(
    _sum_kernel,
    out_shape=jax.ShapeDtypeStruct((_CROWS, D), jnp.float32),
))


_BT = 2000


def _transform_kernel(x_ref, w_ref, b_ref, y_ref):
    i = pl.program_id(1)
    y = jnp.dot(x_ref[...], w_ref[0],
                preferred_element_type=jnp.float32,
                precision=lax.Precision.HIGHEST)
    scale = jnp.where(i == 0, jnp.float32(1.0), jnp.float32(0.0))
    y_ref[0] = y + b_ref[...] * scale


@jax.jit
def _transform(x, wcat, b2d):
    return pl.pallas_call(
        _transform_kernel,
        grid=(N // _BT, R + 1),
        in_specs=[
            pl.BlockSpec((_BT, D), lambda nb, i: (nb, 0)),
            pl.BlockSpec((1, D, D), lambda nb, i: (i, 0, 0)),
            pl.BlockSpec((1, D), lambda nb, i: (0, 0)),
        ],
        out_specs=pl.BlockSpec((1, _BT, D), lambda nb, i: (i, nb, 0)),
        out_shape=jax.ShapeDtypeStruct((R + 1, N, D), jnp.float32),
    )(x, wcat, b2d)


def _combine_kernel(y_ref, p_ref, o_ref, *, relu):
    o = y_ref[0] + p_ref[...]
    if relu:
        o = jnp.maximum(o, 0.0)
    o_ref[...] = o


def _make_combine(relu):
    f = pl.pallas_call(
        functools.partial(_combine_kernel, relu=relu),
        grid=(N // _BT,),
        in_specs=[
            pl.BlockSpec((1, _BT, D), lambda nb: (0, nb, 0)),
            pl.BlockSpec((_BT, D), lambda nb: (nb, 0)),
        ],
        out_specs=pl.BlockSpec((_BT, D), lambda nb: (nb, 0)),
        out_shape=jax.ShapeDtypeStruct((N, D), jnp.float32),
    )
    return jax.jit(f)


_combine_relu = _make_combine(True)
_combine_plain = _make_combine(False)


def kernel(x, edge_index, edge_type, W_rel1, W_root1, b1, W_rel2, W_root2, b2):
    si4 = edge_index[0].reshape(NS, NROUND, ROUND, CHUNK)
    di4 = edge_index[1].reshape(NS, NROUND, ROUND, CHUNK)
    ti4 = edge_type.reshape(NS, NROUND, ROUND, CHUNK)

    cntf = _sum_counts(_count(ti4, di4)).reshape(RN)
    g4, s4 = _edge_scale(cntf, ti4, si4, di4)

    wcat1 = jnp.concatenate([W_root1[None], W_rel1], axis=0)
    y1 = _transform(x, wcat1, b1.reshape(1, D))
    p1 = _scatter(y1.reshape(YR, D), g4, di4, s4)
    h = _combine_relu(y1, p1)

    wcat2 = jnp.concatenate([W_root2[None], W_rel2], axis=0)
    y2 = _transform(h, wcat2, b2.reshape(1, D))
    p2 = _scatter(y2.reshape(YR, D), g4, di4, s4)
    return _combine_plain(y2, p2)
